# baseline (speedup 1.0000x reference)
import functools

import jax
import jax.numpy as jnp
from jax import lax
from jax.experimental import pallas as pl
from jax.experimental.pallas import tpu as pltpu

D_MODEL = 2048
E = 16
TOPK = 2
ROWS = TOPK * E
T = 8192
CAP = T // E
TBLK = 512
_INT_MIN = -(2 ** 31)


def _fused_body(c_ref, f_ref, idx_ref, val_ref, afft_ref):
    i = pl.program_id(0)
    afft_ref[:, pl.ds(i * TBLK, TBLK)] = lax.dot_general(
        c_ref[...], f_ref[...], (((1,), (1,)), ((), ())),
        preferred_element_type=jnp.float32)

    @pl.when(i == T // TBLK - 1)
    def _():
        _auction_sort(afft_ref, idx_ref, val_ref)


def _gate_fused(cs, feats, interpret=False):
    return pl.pallas_call(
        _fused_body,
        grid=(T // TBLK,),
        in_specs=[
            pl.BlockSpec((ROWS, D_MODEL), lambda i: (0, 0)),
            pl.BlockSpec((TBLK, D_MODEL), lambda i: (i, 0)),
        ],
        out_specs=(pl.BlockSpec((ROWS, CAP), lambda i: (0, 0)),
                   pl.BlockSpec((ROWS, CAP), lambda i: (0, 0))),
        out_shape=(
            jax.ShapeDtypeStruct((ROWS, CAP), jnp.int32),
            jax.ShapeDtypeStruct((ROWS, CAP), jnp.float32),
        ),
        scratch_shapes=[pltpu.VMEM((ROWS, T), jnp.float32)],
        interpret=interpret,
    )(cs, feats)


def _to_key(aff):
    aff = jnp.where(aff == 0.0, 0.0, aff)
    b = lax.bitcast_convert_type(aff, jnp.int32)
    return jnp.where(b >= 0, b, b ^ jnp.int32(0x7FFFFFFF))


def _from_key(k):
    b = jnp.where(k >= 0, k, k ^ jnp.int32(0x7FFFFFFF))
    return lax.bitcast_convert_type(b, jnp.float32)


def _auction_sort(aff_ref, idx_ref, val_ref):
    aff = aff_ref[...]

    finite = jnp.isfinite(aff)
    aff3 = aff.reshape(TOPK, E, T)
    fin3 = finite.reshape(TOPK, E, T)
    rmin = jnp.min(jnp.where(fin3, aff3, jnp.inf), axis=(1, 2), keepdims=True)
    aff = jnp.where(fin3, aff3, rmin).reshape(ROWS, T)

    skey = _to_key(aff)
    erow = lax.broadcasted_iota(jnp.int32, (TOPK, E, 1), 1)
    tix = lax.broadcasted_iota(jnp.int32, (ROWS, T), 1)

    def select_value_bit(i, thr_u, unkey, r):
        bit = lax.shift_left(jnp.int32(1), 31 - i)
        trial_u = thr_u | bit
        trial_s = trial_u ^ jnp.int32(_INT_MIN)
        cnt = jnp.sum(jnp.where(unkey >= trial_s, 1, 0), axis=1,
                      keepdims=True)
        return jnp.where(cnt >= r, trial_u, thr_u)

    def select_token_bit(i, tstar, tie_t, q):
        bit = lax.shift_left(jnp.int32(1), 12 - i)
        trial = tstar | bit
        cnt = jnp.sum(jnp.where(tie_t < trial, 1, 0), axis=1,
                      keepdims=True)
        return jnp.where(cnt >= q, tstar, trial)

    def cond(state):
        un, _, _ = state
        return jnp.any(un > 0)

    def body(state):
        un, g, r = state
        unb = jnp.broadcast_to(un > 0, (TOPK, E, T)).reshape(ROWS, T)
        avail = r > 0
        avail3 = avail.reshape(TOPK, E, 1)

        sk3 = skey.reshape(TOPK, E, T)
        mk = jnp.where(avail3, sk3, jnp.int32(_INT_MIN))
        bestv = jnp.max(mk, axis=1, keepdims=True)
        cand = jnp.where((mk == bestv) & avail3, erow, jnp.int32(E))
        beste = jnp.min(cand, axis=1, keepdims=True)

        unkey = jnp.where(unb, skey, jnp.int32(_INT_MIN))
        thr_u = lax.fori_loop(
            0, 32, lambda i, a: select_value_bit(i, a, unkey, r),
            jnp.zeros((ROWS, 1), jnp.int32))
        thr_s = thr_u ^ jnp.int32(_INT_MIN)
        s = jnp.sum(jnp.where(unkey > thr_s, 1, 0), axis=1, keepdims=True)
        q = r - s
        tie = unb & (skey == thr_s)
        tie_t = jnp.where(tie, tix, jnp.int32(T))
        tiecnt = jnp.sum(jnp.where(tie, 1, 0), axis=1, keepdims=True)
        tstar = lax.cond(
            jnp.max(tiecnt) <= 1,
            lambda: jnp.min(tie_t, axis=1, keepdims=True),
            lambda: lax.fori_loop(
                0, 13, lambda i, a: select_token_bit(i, a, tie_t, q),
                jnp.zeros((ROWS, 1), jnp.int32)))

        elig = unb & jnp.broadcast_to(avail, (ROWS, T)) & (
            (skey > thr_s) | (tie & (tix <= tstar)))
        beste_b = jnp.broadcast_to(beste, (TOPK, E, T)).reshape(ROWS, T)
        erow_b = jnp.broadcast_to(erow, (TOPK, E, T)).reshape(ROWS, T)
        commit_mat = jnp.where(elig & (erow_b == beste_b), 1, 0)
        commit_t = jnp.max(commit_mat.reshape(TOPK, E, T),
                           axis=1, keepdims=True)
        g = jnp.where(commit_t > 0, beste, g)
        un = un * (1 - commit_t)
        r = r - jnp.sum(commit_mat, axis=1, keepdims=True)
        return un, g, r

    un0 = jnp.ones((TOPK, 1, T), jnp.int32)
    g0 = jnp.zeros((TOPK, 1, T), jnp.int32)
    r0 = jnp.full((ROWS, 1), CAP, jnp.int32)
    _, g, _ = lax.while_loop(cond, body, (un0, g0, r0))

    g_b = jnp.broadcast_to(g, (TOPK, E, T)).reshape(ROWS, T)
    erow_b = jnp.broadcast_to(erow, (TOPK, E, T)).reshape(ROWS, T)
    K = jnp.where(g_b == erow_b, skey, jnp.int32(_INT_MIN))
    P = tix

    lane = tix
    size = 2
    while size <= T:
        stride = size // 2
        while stride >= 1:
            pk = jnp.where((lane & stride) == 0,
                           jnp.roll(K, -stride, axis=1),
                           jnp.roll(K, stride, axis=1))
            pp = jnp.where((lane & stride) == 0,
                           jnp.roll(P, -stride, axis=1),
                           jnp.roll(P, stride, axis=1))
            sgt = (K > pk) | ((K == pk) & (P < pp))
            is_lo = (lane & stride) == 0
            desc = (lane & size) == 0
            keep = (is_lo == sgt) == desc
            K = jnp.where(keep, K, pk)
            P = jnp.where(keep, P, pp)
            stride //= 2
        size *= 2

    idx_ref[...] = P[:, :CAP]
    val_ref[...] = _from_key(K[:, :CAP])


@jax.jit
def kernel(input_features, centroid_0, centroid_1):
    feats = input_features.reshape(-1, input_features.shape[-1])
    cs = jnp.concatenate([centroid_0, centroid_1], axis=0)
    idx, val = _gate_fused(cs, feats)
    return (idx.reshape(TOPK, E, CAP), val.reshape(TOPK, E, CAP))


# speedup vs baseline: 1.0723x; 1.0723x over previous
import functools

import jax
import jax.numpy as jnp
from jax import lax
from jax.experimental import pallas as pl

D_MODEL = 2048
E = 16
TOPK = 2
ROWS = TOPK * E
T = 8192
CAP = T // E
TBLK = 512
_INT_MIN = -(2 ** 31)


def _matmul_body(c_ref, f_ref, o_ref):
    o_ref[...] = lax.dot_general(
        c_ref[...], f_ref[...], (((1,), (1,)), ((), ())),
        preferred_element_type=jnp.float32)


def _affinities(cs, feats, interpret=False):
    return pl.pallas_call(
        _matmul_body,
        grid=(T // TBLK,),
        in_specs=[
            pl.BlockSpec((ROWS, D_MODEL), lambda i: (0, 0)),
            pl.BlockSpec((TBLK, D_MODEL), lambda i: (i, 0)),
        ],
        out_specs=pl.BlockSpec((ROWS, TBLK), lambda i: (0, i)),
        out_shape=jax.ShapeDtypeStruct((ROWS, T), jnp.float32),
        interpret=interpret,
    )(cs, feats)


def _to_key(aff):
    aff = jnp.where(aff == 0.0, 0.0, aff)
    b = lax.bitcast_convert_type(aff, jnp.int32)
    return jnp.where(b >= 0, b, b ^ jnp.int32(0x7FFFFFFF))


def _from_key(k):
    b = jnp.where(k >= 0, k, k ^ jnp.int32(0x7FFFFFFF))
    return lax.bitcast_convert_type(b, jnp.float32)


def _auction_body(aff_ref, idx_ref, val_ref):
    aff = aff_ref[...]

    finite = jnp.isfinite(aff)
    aff3 = aff.reshape(TOPK, E, T)
    fin3 = finite.reshape(TOPK, E, T)
    rmin = jnp.min(jnp.where(fin3, aff3, jnp.inf), axis=(1, 2), keepdims=True)
    aff = jnp.where(fin3, aff3, rmin).reshape(ROWS, T)

    skey = _to_key(aff)
    erow = lax.broadcasted_iota(jnp.int32, (TOPK, E, 1), 1)
    tix = lax.broadcasted_iota(jnp.int32, (ROWS, T), 1)

    def select_value_bit(i, thr_u, unkey, r):
        bit = lax.shift_left(jnp.int32(1), 31 - i)
        trial_u = thr_u | bit
        trial_s = trial_u ^ jnp.int32(_INT_MIN)
        cnt = jnp.sum(jnp.where(unkey >= trial_s, 1, 0), axis=1,
                      keepdims=True)
        return jnp.where(cnt >= r, trial_u, thr_u)

    def select_token_bit(i, tstar, tie_t, q):
        bit = lax.shift_left(jnp.int32(1), 12 - i)
        trial = tstar | bit
        cnt = jnp.sum(jnp.where(tie_t < trial, 1, 0), axis=1,
                      keepdims=True)
        return jnp.where(cnt >= q, tstar, trial)

    def cond(state):
        un, _, _ = state
        return jnp.any(un > 0)

    def body(state):
        un, g, r = state
        unb = jnp.broadcast_to(un > 0, (TOPK, E, T)).reshape(ROWS, T)
        avail = r > 0
        avail3 = avail.reshape(TOPK, E, 1)

        sk3 = skey.reshape(TOPK, E, T)
        mk = jnp.where(avail3, sk3, jnp.int32(_INT_MIN))
        bestv = jnp.max(mk, axis=1, keepdims=True)
        cand = jnp.where((mk == bestv) & avail3, erow, jnp.int32(E))
        beste = jnp.min(cand, axis=1, keepdims=True)

        unkey = jnp.where(unb, skey, jnp.int32(_INT_MIN))
        thr_u = lax.fori_loop(
            0, 32, lambda i, a: select_value_bit(i, a, unkey, r),
            jnp.zeros((ROWS, 1), jnp.int32))
        thr_s = thr_u ^ jnp.int32(_INT_MIN)
        s = jnp.sum(jnp.where(unkey > thr_s, 1, 0), axis=1, keepdims=True)
        q = r - s
        tie = unb & (skey == thr_s)
        tie_t = jnp.where(tie, tix, jnp.int32(T))
        tiecnt = jnp.sum(jnp.where(tie, 1, 0), axis=1, keepdims=True)
        tstar = lax.cond(
            jnp.max(tiecnt) <= 1,
            lambda: jnp.min(tie_t, axis=1, keepdims=True),
            lambda: lax.fori_loop(
                0, 13, lambda i, a: select_token_bit(i, a, tie_t, q),
                jnp.zeros((ROWS, 1), jnp.int32)))

        elig = unb & jnp.broadcast_to(avail, (ROWS, T)) & (
            (skey > thr_s) | (tie & (tix <= tstar)))
        beste_b = jnp.broadcast_to(beste, (TOPK, E, T)).reshape(ROWS, T)
        erow_b = jnp.broadcast_to(erow, (TOPK, E, T)).reshape(ROWS, T)
        commit_mat = jnp.where(elig & (erow_b == beste_b), 1, 0)
        commit_t = jnp.max(commit_mat.reshape(TOPK, E, T),
                           axis=1, keepdims=True)
        g = jnp.where(commit_t > 0, beste, g)
        un = un * (1 - commit_t)
        r = r - jnp.sum(commit_mat, axis=1, keepdims=True)
        return un, g, r

    un0 = jnp.ones((TOPK, 1, T), jnp.int32)
    g0 = jnp.zeros((TOPK, 1, T), jnp.int32)
    r0 = jnp.full((ROWS, 1), CAP, jnp.int32)
    _, g, _ = lax.while_loop(cond, body, (un0, g0, r0))

    g_b = jnp.broadcast_to(g, (TOPK, E, T)).reshape(ROWS, T)
    erow_b = jnp.broadcast_to(erow, (TOPK, E, T)).reshape(ROWS, T)
    K = jnp.where(g_b == erow_b, skey, jnp.int32(_INT_MIN))
    P = tix

    lane = tix
    size = 2
    while size <= T:
        stride = size // 2
        while stride >= 1:
            pk = jnp.where((lane & stride) == 0,
                           jnp.roll(K, -stride, axis=1),
                           jnp.roll(K, stride, axis=1))
            pp = jnp.where((lane & stride) == 0,
                           jnp.roll(P, -stride, axis=1),
                           jnp.roll(P, stride, axis=1))
            sgt = (K > pk) | ((K == pk) & (P < pp))
            is_lo = (lane & stride) == 0
            desc = (lane & size) == 0
            keep = (is_lo == sgt) == desc
            K = jnp.where(keep, K, pk)
            P = jnp.where(keep, P, pp)
            stride //= 2
        size *= 2

    idx_ref[...] = P[:, :CAP]
    val_ref[...] = _from_key(K[:, :CAP])


def _gate(afft, interpret=False):
    return pl.pallas_call(
        _auction_body,
        out_shape=(
            jax.ShapeDtypeStruct((ROWS, CAP), jnp.int32),
            jax.ShapeDtypeStruct((ROWS, CAP), jnp.float32),
        ),
        interpret=interpret,
    )(afft)


@jax.jit
def kernel(input_features, centroid_0, centroid_1):
    feats = input_features.reshape(-1, input_features.shape[-1])
    cs = jnp.concatenate([centroid_0, centroid_1], axis=0)
    afft = _affinities(cs, feats)
    idx, val = _gate(afft)
    return (idx.reshape(TOPK, E, CAP), val.reshape(TOPK, E, CAP))
